# Initial kernel scaffold; baseline (speedup 1.0000x reference)
import functools

import jax
import jax.numpy as jnp
from jax import lax
from jax.experimental import pallas as pl
from jax.experimental.pallas import tpu as pltpu
from jax.experimental.pallas import tpu_sc as plsc

_N_FIELDS = 26
_VOCAB = 100000
_EMB_DIM = 32
_BATCH = 16384

_NC = 2
_NS = 16
_NW = _NC * _NS

_TOTAL_ROWS = _BATCH * _N_FIELDS
_ROWS_PER_W = _TOTAL_ROWS // _NW
_CHUNK = 1024
_N_CHUNKS = _ROWS_PER_W // _CHUNK


@jax.jit
def _run(table_flat, idx_flat):
    mesh = plsc.VectorSubcoreMesh(core_axis_name="c", subcore_axis_name="s")

    @functools.partial(
        pl.kernel,
        out_type=jax.ShapeDtypeStruct((_TOTAL_ROWS, _EMB_DIM), jnp.float32),
        mesh=mesh,
        scratch_types=[
            pltpu.VMEM((_ROWS_PER_W,), jnp.int32),
            pltpu.VMEM((2, _CHUNK, _EMB_DIM), jnp.float32),
            pltpu.SemaphoreType.DMA,
        ],
    )
    def k(table_hbm, idx_hbm, out_hbm, idx_v, rows_v, gsem):
        wid = lax.axis_index("s") * _NC + lax.axis_index("c")
        base = wid * _ROWS_PER_W
        pltpu.sync_copy(idx_hbm.at[pl.ds(base, _ROWS_PER_W)], idx_v)

        def gather(i, buf):
            return pltpu.async_copy(
                table_hbm.at[idx_v.at[pl.ds(i * _CHUNK, _CHUNK)]],
                rows_v.at[buf],
                gsem,
            )

        h = gather(0, 0)
        for i in range(_N_CHUNKS):
            nxt = gather(i + 1, (i + 1) % 2) if i + 1 < _N_CHUNKS else None
            h.wait()
            pltpu.sync_copy(
                rows_v.at[i % 2],
                out_hbm.at[pl.ds(base + i * _CHUNK, _CHUNK)],
            )
            h = nxt

    return k(table_flat, idx_flat)


def kernel(x_sparse, tables):
    table_flat = tables.reshape(_N_FIELDS * _VOCAB, _EMB_DIM)
    offsets = jnp.arange(_N_FIELDS, dtype=jnp.int32) * _VOCAB
    idx_flat = (x_sparse.astype(jnp.int32) + offsets[None, :]).reshape(-1)
    out = _run(table_flat, idx_flat)
    return out.reshape(_BATCH, _N_FIELDS, _EMB_DIM)


# speedup vs baseline: 1.1548x; 1.1548x over previous
import functools

import jax
import jax.numpy as jnp
from jax import lax
from jax.experimental import pallas as pl
from jax.experimental.pallas import tpu as pltpu
from jax.experimental.pallas import tpu_sc as plsc

_N_FIELDS = 26
_VOCAB = 100000
_EMB_DIM = 32
_BATCH = 16384

_NC = 2
_NS = 16
_NW = _NC * _NS

_TOTAL_ROWS = _BATCH * _N_FIELDS
_ROWS_PER_W = _TOTAL_ROWS // _NW
_CHUNK = 1024
_N_CHUNKS = _ROWS_PER_W // _CHUNK


@jax.jit
def _run(table_flat, idx_flat):
    mesh = plsc.VectorSubcoreMesh(core_axis_name="c", subcore_axis_name="s")

    @functools.partial(
        pl.kernel,
        out_type=jax.ShapeDtypeStruct((_TOTAL_ROWS, _EMB_DIM), jnp.float32),
        mesh=mesh,
        scratch_types=[
            pltpu.VMEM((_ROWS_PER_W,), jnp.int32),
            pltpu.VMEM((2, _CHUNK, _EMB_DIM), jnp.float32),
            pltpu.SemaphoreType.DMA,
        ],
        compiler_params=pltpu.CompilerParams(use_tc_tiling_on_sc=False),
    )
    def k(table_hbm, idx_hbm, out_hbm, idx_v, rows_v, gsem):
        wid = lax.axis_index("s") * _NC + lax.axis_index("c")
        base = wid * _ROWS_PER_W
        pltpu.sync_copy(idx_hbm.at[pl.ds(base, _ROWS_PER_W)], idx_v)

        def gather(i, buf):
            return pltpu.async_copy(
                table_hbm.at[idx_v.at[pl.ds(i * _CHUNK, _CHUNK)]],
                rows_v.at[buf],
                gsem,
            )

        h = gather(0, 0)
        for i in range(_N_CHUNKS):
            nxt = gather(i + 1, (i + 1) % 2) if i + 1 < _N_CHUNKS else None
            h.wait()
            pltpu.sync_copy(
                rows_v.at[i % 2],
                out_hbm.at[pl.ds(base + i * _CHUNK, _CHUNK)],
            )
            h = nxt

    return k(table_flat, idx_flat)


def kernel(x_sparse, tables):
    table_flat = tables.reshape(_N_FIELDS * _VOCAB, _EMB_DIM)
    offsets = jnp.arange(_N_FIELDS, dtype=jnp.int32) * _VOCAB
    idx_flat = (x_sparse.astype(jnp.int32) + offsets[None, :]).reshape(-1)
    out = _run(table_flat, idx_flat)
    return out.reshape(_BATCH, _N_FIELDS, _EMB_DIM)


# speedup vs baseline: 7.9602x; 6.8929x over previous
import functools

import jax
import jax.numpy as jnp
from jax import lax
from jax.experimental import pallas as pl
from jax.experimental.pallas import tpu as pltpu
from jax.experimental.pallas import tpu_sc as plsc

_N_FIELDS = 26
_VOCAB = 100000
_EMB_DIM = 32
_BATCH = 16384

_NC = 2
_NS = 16
_NW = _NC * _NS

_ROWS = _N_FIELDS * _EMB_DIM
_ROWS_PER_W = _ROWS // _NW
_QTR = _BATCH // 4


@jax.jit
def _run(tables_t, idx_t):
    mesh = plsc.VectorSubcoreMesh(core_axis_name="c", subcore_axis_name="s")

    @functools.partial(
        pl.kernel,
        out_type=jax.ShapeDtypeStruct((_N_FIELDS, _EMB_DIM, _BATCH), jnp.float32),
        mesh=mesh,
        scratch_types=[
            pltpu.VMEM((_VOCAB,), jnp.float32),
            pltpu.VMEM((_BATCH,), jnp.int32),
            pltpu.VMEM((2, _QTR), jnp.float32),
            pltpu.SemaphoreType.DMA,
            pltpu.SemaphoreType.DMA,
        ],
        compiler_params=pltpu.CompilerParams(
            use_tc_tiling_on_sc=True, needs_layout_passes=False
        ),
    )
    def k(tab_hbm, idx_hbm, out_hbm, row_v, idx_v, out_v, osem0, osem1):
        wid = lax.axis_index("s") * _NC + lax.axis_index("c")
        r0 = wid * _ROWS_PER_W
        osems = (osem0, osem1)

        fprev = jnp.int32(-1)
        pending = [None, None]
        for j in range(_ROWS_PER_W):
            r = r0 + j
            f = r // _EMB_DIM
            d = r % _EMB_DIM

            @pl.when(f != fprev)
            def _():
                pltpu.sync_copy(idx_hbm.at[f], idx_v)
            fprev = f

            pltpu.sync_copy(tab_hbm.at[f, d, :], row_v)

            for q in range(4):
                b = q % 2
                if pending[b] is not None:
                    pending[b].wait()
                    pending[b] = None

                @plsc.parallel_loop(0, _QTR // 16, unroll=8)
                def _(i, q=q, b=b):
                    iv = idx_v[pl.ds(q * _QTR + i * 16, 16)]
                    out_v[b, pl.ds(i * 16, 16)] = plsc.load_gather(row_v, [iv])

                pending[b] = pltpu.async_copy(
                    out_v.at[b], out_hbm.at[f, d, pl.ds(q * _QTR, _QTR)], osems[b])
        for p in pending:
            if p is not None:
                p.wait()

    return k(tables_t, idx_t)


def kernel(x_sparse, tables):
    tables_t = jnp.transpose(tables, (0, 2, 1))
    idx_t = jnp.transpose(x_sparse.astype(jnp.int32), (1, 0))
    out_t = _run(tables_t, idx_t)
    return jnp.transpose(out_t, (2, 0, 1))
